# baseline (device time: 162545 ns/iter reference)
import jax
import jax.numpy as jnp
from jax import lax
from jax.experimental import pallas as pl
from jax.experimental.pallas import tpu as pltpu

N_DEV = 4
XP = 4
FP = 2
WPIECES = 4


def kernel(x, w_mat):
    m_per, k = x.shape
    _, n_per = w_mat.shape
    M = N_DEV * m_per
    xp = m_per // XP
    wp = k // WPIECES

    def body(x_hbm, w_hbm, out_hbm, xg_ref, wb_ref, y_ref, xstage, wstage,
             qstage, amax_ref, xsems, wsems, qsems,
             send_sems, recv_sems, a_send_sems, a_recv_sems):
        my = lax.axis_index("i")
        left = (my - 1) % N_DEV
        right = (my + 1) % N_DEV

        barrier_sem = pltpu.get_barrier_semaphore()
        for nbr in (left, right):
            pl.semaphore_signal(
                barrier_sem, inc=1,
                device_id=(nbr,), device_id_type=pl.DeviceIdType.MESH,
            )

        def mk(sl, sem_idx, target):
            return pltpu.make_async_remote_copy(
                src_ref=xg_ref.at[sl, :],
                dst_ref=xg_ref.at[sl, :],
                send_sem=send_sems.at[sem_idx],
                recv_sem=recv_sems.at[sem_idx],
                device_id=(target,),
                device_id_type=pl.DeviceIdType.MESH,
            )

        def gemm(row_start, rows):
            sl = pl.ds(row_start, rows)
            yj = jnp.dot(
                xg_ref[sl, :], wb_ref[...],
                preferred_element_type=jnp.float32,
            )
            yj = jnp.maximum(yj, 0.0)
            y_ref[sl, :] = yj
            return jnp.max(yj)

        xdmas = [
            pltpu.make_async_copy(
                x_hbm.at[pl.ds(t * xp, xp), :], xstage.at[t % 2], xsems.at[t % 2]
            )
            for t in range(XP)
        ]
        xdmas[0].start()
        hop0r, hop0l = [], []
        for t in range(XP):
            if t + 1 < XP:
                xdmas[t + 1].start()
            xdmas[t].wait()
            psl = pl.ds(my * m_per + t * xp, xp)
            xg_ref[psl, :] = xstage[t % 2].astype(jnp.bfloat16)
            if t == 0:
                pl.semaphore_wait(barrier_sem, 2)
            sr = mk(psl, t, right)
            sl_ = mk(psl, 8 + t, left)
            sr.start()
            sl_.start()
            hop0r.append(sr)
            hop0l.append(sl_)

        wdmas = [
            pltpu.make_async_copy(
                w_hbm.at[pl.ds(t * wp, wp), :], wstage.at[t % 2], wsems.at[t % 2]
            )
            for t in range(WPIECES)
        ]
        wdmas[0].start()
        for t in range(WPIECES):
            if t + 1 < WPIECES:
                wdmas[t + 1].start()
            wdmas[t].wait()
            wb_ref[pl.ds(t * wp, wp), :] = wstage[t % 2].astype(jnp.bfloat16)

        local_amax = gemm(my * m_per, m_per)

        cl = (my - 1) % N_DEV
        cr = (my + 1) % N_DEV
        hop1r, hop1l = [], []
        for t in range(XP):
            hop0r[t].wait()
            if t < FP:
                f = mk(pl.ds(cl * m_per + t * xp, xp), 16 + t, right)
                f.start()
                hop1r.append(f)
            local_amax = jnp.maximum(
                local_amax, gemm(cl * m_per + t * xp, xp))
        for t in range(XP):
            hop0l[t].wait()
            if t >= XP - FP:
                u = t - (XP - FP)
                f = mk(pl.ds(cr * m_per + t * xp, xp), 20 + u, left)
                f.start()
                hop1l.append(f)
            local_amax = jnp.maximum(
                local_amax, gemm(cr * m_per + t * xp, xp))

        d = (my + 2) % N_DEV
        for u in range(FP):
            hop1r[u].wait()
            local_amax = jnp.maximum(
                local_amax, gemm(d * m_per + u * xp, xp))
            hop1l[u].wait()
            local_amax = jnp.maximum(
                local_amax, gemm(d * m_per + (FP + u) * xp, xp))

        amax_ref[pl.ds(my, 1), :] = jnp.full((1, 128), local_amax, jnp.float32)
        amax_rdmas = []
        for off in range(1, N_DEV):
            p = (my + off) % N_DEV
            r = pltpu.make_async_remote_copy(
                src_ref=amax_ref.at[pl.ds(my, 1), :],
                dst_ref=amax_ref.at[pl.ds(my, 1), :],
                send_sem=a_send_sems.at[off - 1],
                recv_sem=a_recv_sems.at[off - 1],
                device_id=(p,),
                device_id_type=pl.DeviceIdType.MESH,
            )
            r.start()
            amax_rdmas.append(r)
        for r in amax_rdmas:
            r.wait()

        gmax = jnp.max(amax_ref[...])
        scale = gmax / 448.0
        inv_scale = 448.0 / gmax

        qdmas = []
        for j in range(N_DEV):
            if j >= 2:
                qdmas[j - 2].wait()
            blk = y_ref[j * m_per:(j + 1) * m_per, :]
            q = (blk * inv_scale).astype(jnp.float8_e4m3fn).astype(jnp.float32)
            qstage[j % 2] = q * scale
            dma = pltpu.make_async_copy(
                qstage.at[j % 2],
                out_hbm.at[pl.ds(j * m_per, m_per), :],
                qsems.at[j % 2],
            )
            dma.start()
            qdmas.append(dma)
        qdmas[2].wait()
        qdmas[3].wait()

    return pl.pallas_call(
        body,
        out_shape=jax.ShapeDtypeStruct((M, n_per), jnp.float32),
        in_specs=[
            pl.BlockSpec(memory_space=pl.ANY),
            pl.BlockSpec(memory_space=pl.ANY),
        ],
        out_specs=pl.BlockSpec(memory_space=pl.ANY),
        scratch_shapes=[
            pltpu.VMEM((M, k), jnp.bfloat16),
            pltpu.VMEM((k, n_per), jnp.bfloat16),
            pltpu.VMEM((M, n_per), jnp.float32),
            pltpu.VMEM((2, m_per // XP, k), jnp.float32),
            pltpu.VMEM((2, k // WPIECES, n_per), jnp.float32),
            pltpu.VMEM((2, m_per, n_per), jnp.float32),
            pltpu.VMEM((N_DEV, 128), jnp.float32),
            pltpu.SemaphoreType.DMA((2,)),
            pltpu.SemaphoreType.DMA((2,)),
            pltpu.SemaphoreType.DMA((2,)),
            pltpu.SemaphoreType.DMA((24,)),
            pltpu.SemaphoreType.DMA((24,)),
            pltpu.SemaphoreType.DMA((N_DEV - 1,)),
            pltpu.SemaphoreType.DMA((N_DEV - 1,)),
        ],
        compiler_params=pltpu.CompilerParams(
            collective_id=0,
            vmem_limit_bytes=64 * 1024 * 1024,
        ),
    )(x, w_mat)


# device time: 157714 ns/iter; 1.0306x vs baseline; 1.0306x over previous
import jax
import jax.numpy as jnp
from jax import lax
from jax.experimental import pallas as pl
from jax.experimental.pallas import tpu as pltpu

N_DEV = 4
XPIECES = 4
WPIECES = 4


def kernel(x, w_mat):
    m_per, k = x.shape
    _, n_per = w_mat.shape
    M = N_DEV * m_per
    xp = m_per // XPIECES
    wp = k // WPIECES

    def body(x_hbm, w_hbm, out_hbm, xg_ref, wb_ref, y_ref, xstage, wstage,
             qstage, amax_ref, xsems, wsems, qsems, send_sems, recv_sems,
             a_send_sems, a_recv_sems):
        my = lax.axis_index("i")
        left = (my - 1) % N_DEV
        right = (my + 1) % N_DEV

        barrier_sem = pltpu.get_barrier_semaphore()
        for nbr in (left, right):
            pl.semaphore_signal(
                barrier_sem, inc=1,
                device_id=(nbr,), device_id_type=pl.DeviceIdType.MESH,
            )

        def mk(sl, sem_idx, target):
            return pltpu.make_async_remote_copy(
                src_ref=xg_ref.at[sl, :],
                dst_ref=xg_ref.at[sl, :],
                send_sem=send_sems.at[sem_idx],
                recv_sem=recv_sems.at[sem_idx],
                device_id=(target,),
                device_id_type=pl.DeviceIdType.MESH,
            )

        def gemm(row_start, rows):
            sl = pl.ds(row_start, rows)
            yj = jnp.dot(
                xg_ref[sl, :], wb_ref[...],
                preferred_element_type=jnp.float32,
            )
            yj = jnp.maximum(yj, 0.0)
            y_ref[sl, :] = yj
            return jnp.max(yj)

        xdmas = [
            pltpu.make_async_copy(
                x_hbm.at[pl.ds(t * xp, xp), :], xstage.at[t % 2], xsems.at[t % 2]
            )
            for t in range(XPIECES)
        ]
        xdmas[0].start()
        hop0 = []
        for t in range(XPIECES):
            if t + 1 < XPIECES:
                xdmas[t + 1].start()
            xdmas[t].wait()
            xg_ref[pl.ds(my * m_per + t * xp, xp), :] = (
                xstage[t % 2].astype(jnp.bfloat16)
            )
            if t == 0:
                pl.semaphore_wait(barrier_sem, 2)
            psl = pl.ds(my * m_per + t * xp, xp)
            sr = mk(psl, t, right)
            sl_ = mk(psl, 4 + t, left)
            sr.start()
            sl_.start()
            hop0.append((sr, sl_))

        wdmas = [
            pltpu.make_async_copy(
                w_hbm.at[pl.ds(t * wp, wp), :], wstage.at[t % 2], wsems.at[t % 2]
            )
            for t in range(WPIECES)
        ]
        wdmas[0].start()
        for t in range(WPIECES):
            if t + 1 < WPIECES:
                wdmas[t + 1].start()
            wdmas[t].wait()
            wb_ref[pl.ds(t * wp, wp), :] = wstage[t % 2].astype(jnp.bfloat16)

        local_amax = gemm(my * m_per, m_per)

        cl = (my - 1) % N_DEV
        cr = (my + 1) % N_DEV
        hop1 = []
        for u in range(2):
            hop0[u][0].wait()
            f = mk(pl.ds(cl * m_per + u * xp, xp), 8 + u, right)
            f.start()
            hop1.append(f)
        hop0[2][0].wait()
        hop0[3][0].wait()
        hop0[0][1].wait()
        hop0[1][1].wait()
        for u in range(2):
            hop0[2 + u][1].wait()
            f = mk(pl.ds(cr * m_per + (2 + u) * xp, xp), 10 + u, left)
            f.start()
            hop1.append(f)

        local_amax = jnp.maximum(local_amax, gemm(cl * m_per, m_per))
        local_amax = jnp.maximum(local_amax, gemm(cr * m_per, m_per))

        d = (my + 2) % N_DEV
        for u in range(2):
            hop1[u].wait()
            local_amax = jnp.maximum(
                local_amax, gemm(d * m_per + u * xp, xp))
            hop1[2 + u].wait()
            local_amax = jnp.maximum(
                local_amax, gemm(d * m_per + (2 + u) * xp, xp))

        amax_ref[pl.ds(my, 1), :] = jnp.full((1, 128), local_amax, jnp.float32)
        amax_rdmas = []
        for off in range(1, N_DEV):
            p = (my + off) % N_DEV
            r = pltpu.make_async_remote_copy(
                src_ref=amax_ref.at[pl.ds(my, 1), :],
                dst_ref=amax_ref.at[pl.ds(my, 1), :],
                send_sem=a_send_sems.at[off - 1],
                recv_sem=a_recv_sems.at[off - 1],
                device_id=(p,),
                device_id_type=pl.DeviceIdType.MESH,
            )
            r.start()
            amax_rdmas.append(r)
        for r in amax_rdmas:
            r.wait()

        gmax = jnp.max(amax_ref[...])
        scale = gmax / 448.0
        inv_scale = 448.0 / gmax
        qdmas = []
        for j in range(N_DEV):
            if j >= 2:
                qdmas[j - 2].wait()
            blk = y_ref[j * m_per:(j + 1) * m_per, :]
            q = (blk * inv_scale).astype(jnp.float8_e4m3fn).astype(jnp.float32)
            qstage[j % 2] = q * scale
            dma = pltpu.make_async_copy(
                qstage.at[j % 2],
                out_hbm.at[pl.ds(j * m_per, m_per), :],
                qsems.at[j % 2],
            )
            dma.start()
            qdmas.append(dma)
        qdmas[2].wait()
        qdmas[3].wait()

    return pl.pallas_call(
        body,
        out_shape=jax.ShapeDtypeStruct((M, n_per), jnp.float32),
        in_specs=[
            pl.BlockSpec(memory_space=pl.ANY),
            pl.BlockSpec(memory_space=pl.ANY),
        ],
        out_specs=pl.BlockSpec(memory_space=pl.ANY),
        scratch_shapes=[
            pltpu.VMEM((M, k), jnp.bfloat16),
            pltpu.VMEM((k, n_per), jnp.bfloat16),
            pltpu.VMEM((M, n_per), jnp.float32),
            pltpu.VMEM((2, m_per // XPIECES, k), jnp.float32),
            pltpu.VMEM((2, k // WPIECES, n_per), jnp.float32),
            pltpu.VMEM((2, m_per, n_per), jnp.float32),
            pltpu.VMEM((N_DEV, 128), jnp.float32),
            pltpu.SemaphoreType.DMA((2,)),
            pltpu.SemaphoreType.DMA((2,)),
            pltpu.SemaphoreType.DMA((2,)),
            pltpu.SemaphoreType.DMA((12,)),
            pltpu.SemaphoreType.DMA((12,)),
            pltpu.SemaphoreType.DMA((N_DEV - 1,)),
            pltpu.SemaphoreType.DMA((N_DEV - 1,)),
        ],
        compiler_params=pltpu.CompilerParams(
            collective_id=0,
            vmem_limit_bytes=64 * 1024 * 1024,
        ),
    )(x, w_mat)
